# baseline (speedup 1.0000x reference)
import functools

import jax
import jax.numpy as jnp
from jax import lax
from jax.experimental import pallas as pl
from jax.experimental.pallas import tpu as pltpu
from jax.experimental.pallas import tpu_sc as plsc

NUM_FACTORS = 128
AGE_BINS = 10
BATCH = 16384
AGE_MIN = 5.0
AGE_MAX = 21.0

NC = 2
NS = 16
NW = NC * NS
ROWS_PER_W = BATCH // NW
NCH = NUM_FACTORS // 16
NCOMBO = 2 * AGE_BINS


def _body(pred_hbm, age_hbm, gen_hbm, means_hbm, stds_hbm, gadj_hbm, w_hbm,
          out_hbm,
          pred_v, age_v, gen_v, means_v, stds_v, gadj_v, w_v,
          a_v, c_v, cidx_v, perm_v, off_lo, off_w, sem, sem_idx, sem_tab):
    wid = lax.axis_index("s") * NC + lax.axis_index("c")
    base = wid * ROWS_PER_W

    cp = pltpu.async_copy(pred_hbm.at[pl.ds(base, ROWS_PER_W), :], pred_v, sem)
    cp_age = pltpu.async_copy(age_hbm.at[pl.ds(base, ROWS_PER_W)], age_v, sem_idx)
    cp_gen = pltpu.async_copy(gen_hbm.at[pl.ds(base, ROWS_PER_W)], gen_v, sem_idx)
    cp_m = pltpu.async_copy(means_hbm, means_v, sem_tab)
    cp_s = pltpu.async_copy(stds_hbm, stds_v, sem_tab)
    cp_g = pltpu.async_copy(gadj_hbm, gadj_v, sem_tab)
    cp_w = pltpu.async_copy(w_hbm, w_v, sem_tab)
    cp_m.wait()
    cp_s.wait()
    cp_g.wait()
    cp_w.wait()

    @plsc.parallel_loop(0, AGE_BINS)
    def tab_body(b):
        ws = [w_v[pl.ds(j * 16, 16)] for j in range(NCH)]
        gw0s = [gadj_v[0, pl.ds(j * 16, 16)] * ws[j] for j in range(NCH)]
        gw1s = [gadj_v[1, pl.ds(j * 16, 16)] * ws[j] for j in range(NCH)]
        as_ = [ws[j] / (stds_v[b, pl.ds(j * 16, 16)] + 1e-8)
               for j in range(NCH)]
        mas = [means_v[b, pl.ds(j * 16, 16)] * as_[j] for j in range(NCH)]
        for j in range(NCH):
            sl = pl.ds(j * 16, 16)
            a_v[b, sl] = as_[j]
            c_v[2 * b, sl] = gw0s[j] - mas[j]
            c_v[2 * b + 1, sl] = gw1s[j] - mas[j]

    inv_range = 1.0 / (AGE_MAX - AGE_MIN)
    cp_age.wait()
    cp_gen.wait()

    @plsc.parallel_loop(0, ROWS_PER_W // 16)
    def bin_body(t):
        sl = pl.ds(t * 16, 16)
        ages = age_v[sl].astype(jnp.float32)
        na = jnp.clip((ages - AGE_MIN) * inv_range, 0.0, 1.0)
        bins = (na * (AGE_BINS - 1)).astype(jnp.int32)
        cidx_v[0, sl] = bins * 2 + gen_v[sl]

    iota = lax.iota(jnp.int32, 16)
    zeros16 = jnp.zeros((16,), jnp.int32)

    off_w[0, pl.ds(0, 16)] = zeros16
    off_w[0, pl.ds(16, 16)] = zeros16

    def cnt_body(t, carry):
        c16 = cidx_v[0, pl.ds(t * 16, 16)]
        rank, last = plsc.scan_count(c16)
        plsc.addupdate_scatter(off_w, [zeros16, c16], rank, mask=last)
        return carry

    lax.fori_loop(0, ROWS_PER_W // 16, cnt_body, 0)

    v0 = off_w[0, pl.ds(0, 16)]
    v1 = off_w[0, pl.ds(16, 16)]
    c0 = plsc.cumsum(v0)
    c1 = plsc.cumsum(v1) + lax.broadcast_in_dim(c0[15], (16,), ())
    e0 = c0 - v0
    e1 = c1 - v1
    off_lo[0, pl.ds(0, 16)] = e0
    off_lo[0, pl.ds(16, 16)] = e1
    off_w[0, pl.ds(0, 16)] = e0
    off_w[0, pl.ds(16, 16)] = e1

    def pos_body(t, carry):
        c16 = cidx_v[0, pl.ds(t * 16, 16)]
        rank, last = plsc.scan_count(c16)
        basep = plsc.load_gather(off_w, [zeros16, c16])
        pos = basep + rank - 1
        plsc.store_scatter(perm_v, [zeros16, pos], iota + t * 16)
        plsc.addupdate_scatter(off_w, [zeros16, c16], rank, mask=last)
        return carry

    lax.fori_loop(0, ROWS_PER_W // 16, pos_body, 0)

    cp.wait()

    def seg_body(k, carry):
        ksp = lax.broadcast_in_dim(k, (16,), ())
        lo = plsc.load_gather(off_lo, [zeros16, ksp])[0]
        hi = plsc.load_gather(off_w, [zeros16, ksp])[0]
        b = lax.shift_right_logical(k, 1)
        aks = [a_v[b, pl.ds(j * 16, 16)] for j in range(NCH)]
        cks = [c_v[k, pl.ds(j * 16, 16)] for j in range(NCH)]

        @plsc.parallel_loop(lo, hi, unroll=4)
        def srow(p):
            src = plsc.load_gather(perm_v, [zeros16,
                                            lax.broadcast_in_dim(p, (16,), ())])
            ps = [plsc.load_gather(pred_v, [src, iota + (j * 16)])
                  for j in range(NCH)]
            outs = [ps[j] * aks[j] + cks[j] for j in range(NCH)]
            for j in range(NCH):
                plsc.store_scatter(pred_v, [src, iota + (j * 16)], outs[j])

        return carry

    lax.fori_loop(0, NCOMBO, seg_body, 0)

    pltpu.sync_copy(pred_v, out_hbm.at[pl.ds(base, ROWS_PER_W), :])


@functools.cache
def _build():
    mesh = plsc.VectorSubcoreMesh(
        core_axis_name="c", subcore_axis_name="s",
        num_cores=NC, num_subcores=NS)
    return pl.kernel(
        _body,
        out_type=jax.ShapeDtypeStruct((BATCH, NUM_FACTORS), jnp.float32),
        mesh=mesh,
        compiler_params=pltpu.CompilerParams(needs_layout_passes=False),
        scratch_types=[
            pltpu.VMEM((ROWS_PER_W, NUM_FACTORS), jnp.float32),
            pltpu.VMEM((ROWS_PER_W,), jnp.int32),
            pltpu.VMEM((ROWS_PER_W,), jnp.int32),
            pltpu.VMEM((AGE_BINS, NUM_FACTORS), jnp.float32),
            pltpu.VMEM((AGE_BINS, NUM_FACTORS), jnp.float32),
            pltpu.VMEM((2, NUM_FACTORS), jnp.float32),
            pltpu.VMEM((NUM_FACTORS,), jnp.float32),
            pltpu.VMEM((AGE_BINS, NUM_FACTORS), jnp.float32),
            pltpu.VMEM((2 * AGE_BINS, NUM_FACTORS), jnp.float32),
            pltpu.VMEM((1, ROWS_PER_W), jnp.int32),
            pltpu.VMEM((1, ROWS_PER_W), jnp.int32),
            pltpu.VMEM((1, 32), jnp.int32),
            pltpu.VMEM((1, 32), jnp.int32),
            pltpu.SemaphoreType.DMA,
            pltpu.SemaphoreType.DMA,
            pltpu.SemaphoreType.DMA,
        ],
    )


def kernel(predictions, age, gender, age_means, age_stds, gender_adjustments,
           norm_weights):
    age = age.astype(jnp.int32)
    gender = gender.astype(jnp.int32)
    return _build()(predictions, age, gender, age_means, age_stds,
                    gender_adjustments, norm_weights)


# speedup vs baseline: 1.0896x; 1.0896x over previous
import functools

import jax
import jax.numpy as jnp
from jax import lax
from jax.experimental import pallas as pl
from jax.experimental.pallas import tpu as pltpu
from jax.experimental.pallas import tpu_sc as plsc

NUM_FACTORS = 128
AGE_BINS = 10
BATCH = 16384
AGE_MIN = 5.0
AGE_MAX = 21.0

NC = 2
NS = 16
NW = NC * NS
ROWS_PER_W = BATCH // NW
NCH = NUM_FACTORS // 16
NCOMBO = 2 * AGE_BINS


def _body(pred_hbm, age_hbm, gen_hbm, means_hbm, stds_hbm, gadj_hbm, w_hbm,
          out_hbm,
          pred_v, age_v, gen_v, means_v, stds_v, gadj_v, w_v,
          a_v, c_v, cidx_v, perm_v, off_lo, off_w, sem, sem_idx, sem_tab):
    wid = lax.axis_index("s") * NC + lax.axis_index("c")
    base = wid * ROWS_PER_W

    cp = pltpu.async_copy(pred_hbm.at[pl.ds(base, ROWS_PER_W), :], pred_v, sem)
    cp_age = pltpu.async_copy(age_hbm.at[pl.ds(base, ROWS_PER_W)], age_v, sem_idx)
    cp_gen = pltpu.async_copy(gen_hbm.at[pl.ds(base, ROWS_PER_W)], gen_v, sem_idx)
    cp_m = pltpu.async_copy(means_hbm, means_v, sem_tab)
    cp_s = pltpu.async_copy(stds_hbm, stds_v, sem_tab)
    cp_g = pltpu.async_copy(gadj_hbm, gadj_v, sem_tab)
    cp_w = pltpu.async_copy(w_hbm, w_v, sem_tab)
    cp_m.wait()
    cp_s.wait()
    cp_g.wait()
    cp_w.wait()

    @plsc.parallel_loop(0, AGE_BINS)
    def tab_body(b):
        ws = [w_v[pl.ds(j * 16, 16)] for j in range(NCH)]
        gw0s = [gadj_v[0, pl.ds(j * 16, 16)] * ws[j] for j in range(NCH)]
        gw1s = [gadj_v[1, pl.ds(j * 16, 16)] * ws[j] for j in range(NCH)]
        as_ = [ws[j] / (stds_v[b, pl.ds(j * 16, 16)] + 1e-8)
               for j in range(NCH)]
        mas = [means_v[b, pl.ds(j * 16, 16)] * as_[j] for j in range(NCH)]
        for j in range(NCH):
            sl = pl.ds(j * 16, 16)
            a_v[b, sl] = as_[j]
            c_v[2 * b, sl] = gw0s[j] - mas[j]
            c_v[2 * b + 1, sl] = gw1s[j] - mas[j]

    inv_range = 1.0 / (AGE_MAX - AGE_MIN)
    cp_age.wait()
    cp_gen.wait()

    @plsc.parallel_loop(0, ROWS_PER_W // 16)
    def bin_body(t):
        sl = pl.ds(t * 16, 16)
        ages = age_v[sl].astype(jnp.float32)
        na = jnp.clip((ages - AGE_MIN) * inv_range, 0.0, 1.0)
        bins = (na * (AGE_BINS - 1)).astype(jnp.int32)
        cidx_v[0, sl] = bins * 2 + gen_v[sl]

    iota = lax.iota(jnp.int32, 16)
    zeros16 = jnp.zeros((16,), jnp.int32)

    off_w[0, pl.ds(0, 16)] = zeros16
    off_w[0, pl.ds(16, 16)] = zeros16

    def cnt_body(t, carry):
        c16 = cidx_v[0, pl.ds(t * 16, 16)]
        rank, last = plsc.scan_count(c16)
        plsc.addupdate_scatter(off_w, [zeros16, c16], rank, mask=last)
        return carry

    lax.fori_loop(0, ROWS_PER_W // 16, cnt_body, 0)

    v0 = off_w[0, pl.ds(0, 16)]
    v1 = off_w[0, pl.ds(16, 16)]
    c0 = plsc.cumsum(v0)
    c1 = plsc.cumsum(v1) + lax.broadcast_in_dim(c0[15], (16,), ())
    e0 = c0 - v0
    e1 = c1 - v1
    off_lo[0, pl.ds(0, 16)] = e0
    off_lo[0, pl.ds(16, 16)] = e1
    off_w[0, pl.ds(0, 16)] = e0
    off_w[0, pl.ds(16, 16)] = e1

    def pos_body(t, carry):
        c16 = cidx_v[0, pl.ds(t * 16, 16)]
        rank, last = plsc.scan_count(c16)
        basep = plsc.load_gather(off_w, [zeros16, c16])
        pos = basep + rank - 1
        plsc.store_scatter(perm_v, [zeros16, pos], iota + t * 16)
        plsc.addupdate_scatter(off_w, [zeros16, c16], rank, mask=last)
        return carry

    lax.fori_loop(0, ROWS_PER_W // 16, pos_body, 0)

    cp.wait()

    def seg_body(k, carry):
        ksp = lax.broadcast_in_dim(k, (16,), ())
        lo = plsc.load_gather(off_lo, [zeros16, ksp])[0]
        hi = plsc.load_gather(off_w, [zeros16, ksp])[0]
        b = lax.shift_right_logical(k, 1)
        aks = [a_v[b, pl.ds(j * 16, 16)] for j in range(NCH)]
        cks = [c_v[k, pl.ds(j * 16, 16)] for j in range(NCH)]

        @plsc.parallel_loop(lo, hi)
        def srow(p):
            src = plsc.load_gather(perm_v, [zeros16,
                                            lax.broadcast_in_dim(p, (16,), ())])
            ps = [plsc.load_gather(pred_v, [src, iota + (j * 16)])
                  for j in range(NCH)]
            outs = [ps[j] * aks[j] + cks[j] for j in range(NCH)]
            for j in range(NCH):
                plsc.store_scatter(pred_v, [src, iota + (j * 16)], outs[j])

        return carry

    lax.fori_loop(0, NCOMBO, seg_body, 0)

    pltpu.sync_copy(pred_v, out_hbm.at[pl.ds(base, ROWS_PER_W), :])


@functools.cache
def _build():
    mesh = plsc.VectorSubcoreMesh(
        core_axis_name="c", subcore_axis_name="s",
        num_cores=NC, num_subcores=NS)
    return pl.kernel(
        _body,
        out_type=jax.ShapeDtypeStruct((BATCH, NUM_FACTORS), jnp.float32),
        mesh=mesh,
        compiler_params=pltpu.CompilerParams(needs_layout_passes=False),
        scratch_types=[
            pltpu.VMEM((ROWS_PER_W, NUM_FACTORS), jnp.float32),
            pltpu.VMEM((ROWS_PER_W,), jnp.int32),
            pltpu.VMEM((ROWS_PER_W,), jnp.int32),
            pltpu.VMEM((AGE_BINS, NUM_FACTORS), jnp.float32),
            pltpu.VMEM((AGE_BINS, NUM_FACTORS), jnp.float32),
            pltpu.VMEM((2, NUM_FACTORS), jnp.float32),
            pltpu.VMEM((NUM_FACTORS,), jnp.float32),
            pltpu.VMEM((AGE_BINS, NUM_FACTORS), jnp.float32),
            pltpu.VMEM((2 * AGE_BINS, NUM_FACTORS), jnp.float32),
            pltpu.VMEM((1, ROWS_PER_W), jnp.int32),
            pltpu.VMEM((1, ROWS_PER_W), jnp.int32),
            pltpu.VMEM((1, 32), jnp.int32),
            pltpu.VMEM((1, 32), jnp.int32),
            pltpu.SemaphoreType.DMA,
            pltpu.SemaphoreType.DMA,
            pltpu.SemaphoreType.DMA,
        ],
    )


def kernel(predictions, age, gender, age_means, age_stds, gender_adjustments,
           norm_weights):
    age = age.astype(jnp.int32)
    gender = gender.astype(jnp.int32)
    return _build()(predictions, age, gender, age_means, age_stds,
                    gender_adjustments, norm_weights)
